# baseline (speedup 1.0000x reference)
import functools

import jax
import jax.numpy as jnp
from jax import lax
from jax.experimental import pallas as pl
from jax.experimental.pallas import tpu as pltpu
from jax.experimental.pallas import tpu_sc as plsc

_B, _L, _V, _P, _T, _D = 1024, 512, 100000, 512, 2, 128
_N = _B * _L
_EPS = 1e-12

_K = 4
_NS = _N // _K
_NW = 32
_TOK_W = _NS // _NW
_C = 256
_NCH = _TOK_W // _C


def _gather_body(k, ids_hbm, wemb_hbm, out_hbm, idx_v, rows_a, rows_b,
                 gsem_a, gsem_b, wsem_a, wsem_b):
    c = lax.axis_index("c")
    s = lax.axis_index("s")
    wid = s * 2 + c
    base_w = wid * _TOK_W
    base_ids = k * _NS + base_w

    def pair_body(p, carry):
        g0 = 2 * p
        off0 = g0 * _C
        gm0 = 2 * lax.rem(p, 2)

        @pl.when(lax.rem(p, 2) == 0)
        def _():
            row0 = pl.multiple_of((base_ids + off0) // 128, 8)
            pltpu.sync_copy(ids_hbm.at[pl.ds(row0, 8)], idx_v)

        def start(rows_v, gm, gsem):
            return [
                pltpu.async_copy(
                    wemb_hbm.at[idx_v.at[gm * 2 + j]],
                    rows_v.at[pl.ds(j * 128, 128)],
                    gsem,
                )
                for j in range(_C // 128)
            ]

        cps_a = start(rows_a, gm0, gsem_a)
        cps_b = start(rows_b, gm0 + 1, gsem_b)
        for cp in cps_a:
            cp.wait()
        wb_a = pltpu.async_copy(rows_a, out_hbm.at[pl.ds(base_w + off0, _C)],
                                wsem_a)
        for cp in cps_b:
            cp.wait()
        wb_b = pltpu.async_copy(rows_b,
                                out_hbm.at[pl.ds(base_w + off0 + _C, _C)],
                                wsem_b)
        wb_a.wait()
        wb_b.wait()
        return carry

    lax.fori_loop(0, _NCH // 2, pair_body, 0)


def _make_sc_gather(k):
    return pl.kernel(
        functools.partial(_gather_body, k),
        out_type=jax.ShapeDtypeStruct((_NS, _D), jnp.float32),
        mesh=plsc.VectorSubcoreMesh(core_axis_name="c", subcore_axis_name="s"),
        compiler_params=pltpu.CompilerParams(needs_layout_passes=False),
        scratch_types=[
            pltpu.VMEM((8, 128), jnp.int32),
            pltpu.VMEM((_C, _D), jnp.float32),
            pltpu.VMEM((_C, _D), jnp.float32),
            pltpu.SemaphoreType.DMA,
            pltpu.SemaphoreType.DMA,
            pltpu.SemaphoreType.DMA,
            pltpu.SemaphoreType.DMA,
        ],
    )


_sc_gathers = [_make_sc_gather(k) for k in range(_K)]


_SPS = 16
_BLK = _SPS * _L
_STEPS = _NS // _BLK


def _ln_first_body(x_ref, ttf_ref, pos_ref, temb_ref, gam_ref, bet_ref,
                   o_ref):
    _ln_compute(x_ref, ttf_ref, pos_ref, temb_ref, gam_ref, bet_ref, o_ref)


def _ln_chain_body(buf_ref, x_ref, ttf_ref, pos_ref, temb_ref, gam_ref,
                   bet_ref, o_ref):
    del buf_ref
    _ln_compute(x_ref, ttf_ref, pos_ref, temb_ref, gam_ref, bet_ref, o_ref)


def _ln_compute(x_ref, ttf_ref, pos_ref, temb_ref, gam_ref, bet_ref, o_ref):
    x = x_ref[...]
    ttf = ttf_ref[...]
    t0 = temb_ref[0:1, :]
    t1 = temb_ref[1:2, :]
    x = x + pos_ref[...] + t0 + ttf * (t1 - t0)
    mean = jnp.mean(x, axis=-1, keepdims=True)
    xc = x - mean
    var = jnp.mean(xc * xc, axis=-1, keepdims=True)
    inv = lax.rsqrt(var + _EPS)
    o_ref[...] = xc * inv * gam_ref[...] + bet_ref[...]


def _make_tc_ln(k):
    common_in_specs = [
        pl.BlockSpec((_BLK, _D), lambda i: (i, 0)),
        pl.BlockSpec((_BLK, 1), lambda i, k=k: (k * _STEPS + i, 0)),
        pl.BlockSpec((_BLK, _D), lambda i: (0, 0)),
        pl.BlockSpec((_T, _D), lambda i: (0, 0)),
        pl.BlockSpec((1, _D), lambda i: (0, 0)),
        pl.BlockSpec((1, _D), lambda i: (0, 0)),
    ]
    out_spec = pl.BlockSpec((_BLK, _D), lambda i, k=k: (k * _STEPS + i, 0))
    out_shape = jax.ShapeDtypeStruct((_N, _D), jnp.float32)
    if k == 0:
        return pl.pallas_call(
            _ln_first_body,
            grid=(_STEPS,),
            in_specs=common_in_specs,
            out_specs=out_spec,
            out_shape=out_shape,
        )
    return pl.pallas_call(
        _ln_chain_body,
        grid=(_STEPS,),
        in_specs=[pl.BlockSpec(memory_space=pl.ANY)] + common_in_specs,
        out_specs=out_spec,
        out_shape=out_shape,
        input_output_aliases={0: 0},
    )


_tc_lns = [_make_tc_ln(k) for k in range(_K)]


def kernel(input_ids, token_type_ids, word_emb, pos_emb, type_emb,
           ln_gamma, ln_beta):
    ids = input_ids.reshape(-1).astype(jnp.int32).reshape(_N // 128, 128)
    ttf = token_type_ids.reshape(_N, 1).astype(jnp.float32)
    wemb = word_emb.astype(jnp.float32)
    pos_t = jnp.tile(pos_emb.astype(jnp.float32), (_SPS, 1))
    temb = type_emb.astype(jnp.float32)
    gam = ln_gamma.astype(jnp.float32).reshape(1, _D)
    bet = ln_beta.astype(jnp.float32).reshape(1, _D)

    rows = [_sc_gathers[k](ids, wemb) for k in range(_K)]
    out = _tc_lns[0](rows[0], ttf, pos_t, temb, gam, bet)
    for k in range(1, _K):
        out = _tc_lns[k](out, rows[k], ttf, pos_t, temb, gam, bet)
    return out.reshape(_B, _L, _D)


# speedup vs baseline: 1.0095x; 1.0095x over previous
import functools

import jax
import jax.numpy as jnp
from jax import lax
from jax.experimental import pallas as pl
from jax.experimental.pallas import tpu as pltpu
from jax.experimental.pallas import tpu_sc as plsc

_B, _L, _V, _P, _T, _D = 1024, 512, 100000, 512, 2, 128
_N = _B * _L
_EPS = 1e-12

_K = 4
_NS = _N // _K
_NW = 32
_TOK_W = _NS // _NW
_C = 256
_NCH = _TOK_W // _C


def _gather_body(k, ids_hbm, wemb_hbm, out_hbm, idx_v, rows_a, rows_b,
                 gsem_a, gsem_b, wsem_a, wsem_b):
    c = lax.axis_index("c")
    s = lax.axis_index("s")
    wid = s * 2 + c
    base_w = wid * _TOK_W
    base_ids = k * _NS + base_w

    def pair_body(p, carry):
        g0 = 2 * p
        off0 = g0 * _C
        gm0 = 2 * lax.rem(p, 2)

        @pl.when(lax.rem(p, 2) == 0)
        def _():
            row0 = pl.multiple_of((base_ids + off0) // 128, 8)
            pltpu.sync_copy(ids_hbm.at[pl.ds(row0, 8)], idx_v)

        def start(rows_v, gm, gsem):
            return [
                pltpu.async_copy(
                    wemb_hbm.at[idx_v.at[gm * 2 + j]],
                    rows_v.at[pl.ds(j * 128, 128)],
                    gsem,
                )
                for j in range(_C // 128)
            ]

        cps_a = start(rows_a, gm0, gsem_a)
        cps_b = start(rows_b, gm0 + 1, gsem_b)
        for cp in cps_a:
            cp.wait()
        wb_a = pltpu.async_copy(rows_a, out_hbm.at[pl.ds(base_w + off0, _C)],
                                wsem_a)
        for cp in cps_b:
            cp.wait()
        wb_b = pltpu.async_copy(rows_b,
                                out_hbm.at[pl.ds(base_w + off0 + _C, _C)],
                                wsem_b)
        wb_a.wait()
        wb_b.wait()
        return carry

    lax.fori_loop(0, _NCH // 2, pair_body, 0)


def _make_sc_gather(k):
    return pl.kernel(
        functools.partial(_gather_body, k),
        out_type=jax.ShapeDtypeStruct((_NS, _D), jnp.float32),
        mesh=plsc.VectorSubcoreMesh(core_axis_name="c", subcore_axis_name="s"),
        compiler_params=pltpu.CompilerParams(needs_layout_passes=False),
        scratch_types=[
            pltpu.VMEM((8, 128), jnp.int32),
            pltpu.VMEM((_C, _D), jnp.float32),
            pltpu.VMEM((_C, _D), jnp.float32),
            pltpu.SemaphoreType.DMA,
            pltpu.SemaphoreType.DMA,
            pltpu.SemaphoreType.DMA,
            pltpu.SemaphoreType.DMA,
        ],
    )


_sc_gathers = [_make_sc_gather(k) for k in range(_K)]


_SPS = 16
_BLK = _SPS * _L
_STEPS = _NS // _BLK


def _ln_first_body(x_ref, ttf_ref, pos_ref, temb_ref, gam_ref, bet_ref,
                   o_ref):
    _ln_compute(x_ref, ttf_ref, pos_ref, temb_ref, gam_ref, bet_ref, o_ref)


def _ln_chain_body(buf_ref, x_ref, ttf_ref, pos_ref, temb_ref, gam_ref,
                   bet_ref, o_ref):
    del buf_ref
    _ln_compute(x_ref, ttf_ref, pos_ref, temb_ref, gam_ref, bet_ref, o_ref)


def _ln_compute(x_ref, ttf_ref, pos_ref, temb_ref, gam_ref, bet_ref, o_ref):
    x = x_ref[...].reshape(_SPS, _L, _D)
    ttf = ttf_ref[...]
    t0 = temb_ref[0:1, :]
    t1 = temb_ref[1:2, :]
    x = x + pos_ref[...] + t0 + ttf * (t1 - t0)
    mean = jnp.mean(x, axis=-1, keepdims=True)
    xc = x - mean
    var = jnp.mean(xc * xc, axis=-1, keepdims=True)
    inv = lax.rsqrt(var + _EPS)
    o_ref[...] = xc * inv * gam_ref[...] + bet_ref[...]


def _make_tc_ln(k):
    common_in_specs = [
        pl.BlockSpec((_BLK, _D), lambda i: (i, 0)),
        pl.BlockSpec((_SPS, _L, 1), lambda i, k=k: (k * _STEPS + i, 0, 0)),
        pl.BlockSpec((1, _L, _D), lambda i: (0, 0, 0)),
        pl.BlockSpec((_T, _D), lambda i: (0, 0)),
        pl.BlockSpec((1, _D), lambda i: (0, 0)),
        pl.BlockSpec((1, _D), lambda i: (0, 0)),
    ]
    out_spec = pl.BlockSpec((_SPS, _L, _D),
                            lambda i, k=k: (k * _STEPS + i, 0, 0))
    out_shape = jax.ShapeDtypeStruct((_B, _L, _D), jnp.float32)
    if k == 0:
        return pl.pallas_call(
            _ln_first_body,
            grid=(_STEPS,),
            in_specs=common_in_specs,
            out_specs=out_spec,
            out_shape=out_shape,
        )
    return pl.pallas_call(
        _ln_chain_body,
        grid=(_STEPS,),
        in_specs=[pl.BlockSpec(memory_space=pl.ANY)] + common_in_specs,
        out_specs=out_spec,
        out_shape=out_shape,
        input_output_aliases={0: 0},
    )


_tc_lns = [_make_tc_ln(k) for k in range(_K)]


def kernel(input_ids, token_type_ids, word_emb, pos_emb, type_emb,
           ln_gamma, ln_beta):
    ids = input_ids.reshape(-1).astype(jnp.int32).reshape(_N // 128, 128)
    ttf = token_type_ids.reshape(_B, _L, 1).astype(jnp.float32)
    wemb = word_emb.astype(jnp.float32)
    pos3 = pos_emb.astype(jnp.float32).reshape(1, _L, _D)
    temb = type_emb.astype(jnp.float32)
    gam = ln_gamma.astype(jnp.float32).reshape(1, _D)
    bet = ln_beta.astype(jnp.float32).reshape(1, _D)

    rows = [_sc_gathers[k](ids, wemb) for k in range(_K)]
    out = _tc_lns[0](rows[0], ttf, pos3, temb, gam, bet)
    for k in range(1, _K):
        out = _tc_lns[k](out, rows[k], ttf, pos3, temb, gam, bet)
    return out


# speedup vs baseline: 1.0217x; 1.0120x over previous
import functools

import jax
import jax.numpy as jnp
from jax import lax
from jax.experimental import pallas as pl
from jax.experimental.pallas import tpu as pltpu
from jax.experimental.pallas import tpu_sc as plsc

_B, _L, _V, _P, _T, _D = 1024, 512, 100000, 512, 2, 128
_N = _B * _L
_EPS = 1e-12

_NW = 32
_TOK_W = _N // _NW
_C = 256
_NCH = _TOK_W // _C


def _gather_body(ids_hbm, wemb_hbm, out_hbm, idx_v, rows_a, rows_b, sem):
    c = lax.axis_index("c")
    s = lax.axis_index("s")
    wid = s * 2 + c
    base_w = wid * _TOK_W

    def chunk_body(g, carry):
        base = base_w + g * _C
        gm = lax.rem(g, 4)

        @pl.when(gm == 0)
        def _():
            row0 = pl.multiple_of(base // 128, 8)
            pltpu.sync_copy(ids_hbm.at[pl.ds(row0, 8)], idx_v)

        def run(rows_v):
            cps = [
                pltpu.async_copy(
                    wemb_hbm.at[idx_v.at[gm * 2 + j]],
                    rows_v.at[pl.ds(j * 128, 128)],
                    sem,
                )
                for j in range(_C // 128)
            ]
            for cp in cps:
                cp.wait()
            pltpu.sync_copy(rows_v, out_hbm.at[pl.ds(base, _C)])

        @pl.when(lax.rem(g, 2) == 0)
        def _():
            run(rows_a)

        @pl.when(lax.rem(g, 2) == 1)
        def _():
            run(rows_b)

        return carry

    lax.fori_loop(0, _NCH, chunk_body, 0)


_sc_gather = pl.kernel(
    _gather_body,
    out_type=jax.ShapeDtypeStruct((_N, _D), jnp.float32),
    mesh=plsc.VectorSubcoreMesh(core_axis_name="c", subcore_axis_name="s"),
    compiler_params=pltpu.CompilerParams(needs_layout_passes=False),
    scratch_types=[
        pltpu.VMEM((8, 128), jnp.int32),
        pltpu.VMEM((_C, _D), jnp.float32),
        pltpu.VMEM((_C, _D), jnp.float32),
        pltpu.SemaphoreType.DMA,
    ],
)


_SPS = 16
_BLK = _SPS * _L


def _ln_body(x_ref, ttf_ref, pos_ref, temb_ref, gam_ref, bet_ref, o_ref):
    x = x_ref[...]
    ttf = ttf_ref[...]
    t0 = temb_ref[0:1, :]
    t1 = temb_ref[1:2, :]
    x = x + pos_ref[...] + t0 + ttf * (t1 - t0)
    mean = jnp.mean(x, axis=-1, keepdims=True)
    xc = x - mean
    var = jnp.mean(xc * xc, axis=-1, keepdims=True)
    inv = lax.rsqrt(var + _EPS)
    o_ref[...] = xc * inv * gam_ref[...] + bet_ref[...]


_tc_ln = pl.pallas_call(
    _ln_body,
    grid=(_B // _SPS,),
    in_specs=[
        pl.BlockSpec((_BLK, _D), lambda i: (i, 0)),
        pl.BlockSpec((_BLK, 1), lambda i: (i, 0)),
        pl.BlockSpec((_BLK, _D), lambda i: (0, 0)),
        pl.BlockSpec((_T, _D), lambda i: (0, 0)),
        pl.BlockSpec((1, _D), lambda i: (0, 0)),
        pl.BlockSpec((1, _D), lambda i: (0, 0)),
    ],
    out_specs=pl.BlockSpec((_BLK, _D), lambda i: (i, 0)),
    out_shape=jax.ShapeDtypeStruct((_N, _D), jnp.float32),
)


def kernel(input_ids, token_type_ids, word_emb, pos_emb, type_emb,
           ln_gamma, ln_beta):
    ids = input_ids.reshape(-1).astype(jnp.int32).reshape(_N // 128, 128)
    ttf = token_type_ids.reshape(_N, 1).astype(jnp.float32)
    rows = _sc_gather(ids, word_emb.astype(jnp.float32))
    pos_t = jnp.tile(pos_emb.astype(jnp.float32), (_SPS, 1))
    out = _tc_ln(rows, ttf, pos_t, type_emb.astype(jnp.float32),
                 ln_gamma.astype(jnp.float32).reshape(1, _D),
                 ln_beta.astype(jnp.float32).reshape(1, _D))
    return out.reshape(_B, _L, _D)
